# baseline (speedup 1.0000x reference)
import math

import jax
import jax.numpy as jnp
from jax.experimental import pallas as pl

N = 10000
H = 64
TEMB = 64
B = 8

_LOG1E4 = math.log(10000.0)

_SEGS = (
    ('t1w', 64, 128), ('t1b', 1, 128), ('t2w', 128, 64), ('t2b', 1, 64),
    ('to1w', 7, 32), ('to1b', 1, 32), ('to2w', 32, 32), ('to2b', 1, 32),
    ('st1w', 2, 16), ('st1b', 1, 16), ('st2w', 16, 16), ('st2b', 1, 16),
    ('su1w', 3, 16), ('su1b', 1, 16), ('su2w', 16, 16), ('su2b', 1, 16),
    ('c1w', 64, 64), ('c1b', 1, 64), ('c2w', 64, 64), ('c2b', 1, 64),
    ('wv', 64, 128), ('wow', 128, 64), ('wob', 1, 64),
    ('new', 12, 64), ('neb', 1, 64),
    ('np1w', 64, 128), ('np1b', 1, 128), ('np2w', 128, 12), ('np2b', 1, 12),
    ('pp1w', 64, 64), ('pp1b', 1, 64), ('pp2w', 64, 3), ('pp2b', 1, 3),
)
_OFF = {}
_rows = 0
for _nm, _r, _c in _SEGS:
    _OFF[_nm] = _rows
    _rows += _r + (-_r) % 8
_PACKED_ROWS = _rows


def _silu(v):
    return v * jax.nn.sigmoid(v)


def _mm(a, b):
    return jax.lax.dot_general(a, b, (((1,), (0,)), ((), ())),
                               preferred_element_type=jnp.float32)


def _body(x_ref, batch_ref, cin_ref, pk_ref, node_out_ref, pos_out_ref):
    seg = {nm: pk_ref[_OFF[nm]:_OFF[nm] + r, :c] for nm, r, c in _SEGS}

    half = TEMB // 2
    freq = jnp.exp(jax.lax.broadcasted_iota(jnp.int32, (1, half), 1)
                   .astype(jnp.float32) * (-_LOG1E4 / (half - 1)))
    ang = cin_ref[:, 0:1] * freq
    s, c = jnp.sin(ang), jnp.cos(ang)
    te_h = _silu(_mm(s, seg['t1w'][:half, :]) + _mm(c, seg['t1w'][half:, :])
                 + seg['t1b'])
    temb = _mm(te_h, seg['t2w']) + seg['t2b']

    te = _mm(_silu(_mm(cin_ref[:, 1:8], seg['to1w']) + seg['to1b']),
             seg['to2w']) + seg['to2b']
    se = _mm(_silu(_mm(cin_ref[:, 8:10], seg['st1w']) + seg['st1b']),
             seg['st2w']) + seg['st2b']
    ue = _mm(_silu(_mm(cin_ref[:, 10:13], seg['su1w']) + seg['su1b']),
             seg['su2w']) + seg['su2b']
    ce_h = _silu(_mm(te, seg['c1w'][0:32, :]) + _mm(se, seg['c1w'][32:48, :])
                 + _mm(ue, seg['c1w'][48:64, :]) + seg['c1b'])
    cond = _mm(ce_h, seg['c2w']) + seg['c2b']

    attn = _mm(_mm(cond, seg['wv']), seg['wow']) + seg['wob']
    delta = attn + _silu(temb)

    onehot = (batch_ref[...] ==
              jax.lax.broadcasted_iota(jnp.int32, (N, B), 1)
              ).astype(jnp.float32)
    u = _mm(onehot, delta)
    h = _mm(x_ref[...], seg['new']) + seg['neb']
    h = h + u
    h = h + u
    h = h + u
    h = h + u
    a = _silu(_mm(h, seg['np1w']) + seg['np1b'])
    node_out_ref[...] = _mm(a, seg['np2w']) + seg['np2b']
    g = _silu(_mm(h, seg['pp1w']) + seg['pp1b'])
    pos_out_ref[...] = _mm(g, seg['pp2w']) + seg['pp2b']


def kernel(x, edge_index, edge_attr, pos, t, topo_cond, stab_cond, sust_cond,
           batch, params):
    del edge_index, edge_attr, pos
    p = params
    row = lambda b: b.reshape(1, -1)
    vals = {
        't1w': p['time1']['w'], 't1b': row(p['time1']['b']),
        't2w': p['time2']['w'], 't2b': row(p['time2']['b']),
        'to1w': p['topo1']['w'], 'to1b': row(p['topo1']['b']),
        'to2w': p['topo2']['w'], 'to2b': row(p['topo2']['b']),
        'st1w': p['stab1']['w'], 'st1b': row(p['stab1']['b']),
        'st2w': p['stab2']['w'], 'st2b': row(p['stab2']['b']),
        'su1w': p['sust1']['w'], 'su1b': row(p['sust1']['b']),
        'su2w': p['sust2']['w'], 'su2b': row(p['sust2']['b']),
        'c1w': p['comb1']['w'], 'c1b': row(p['comb1']['b']),
        'c2w': p['comb2']['w'], 'c2b': row(p['comb2']['b']),
        'wv': p['Wv'], 'wow': p['Wo']['w'], 'wob': row(p['Wo']['b']),
        'new': p['node_emb']['w'], 'neb': row(p['node_emb']['b']),
        'np1w': p['np1']['w'], 'np1b': row(p['np1']['b']),
        'np2w': p['np2']['w'], 'np2b': row(p['np2']['b']),
        'pp1w': p['pp1']['w'], 'pp1b': row(p['pp1']['b']),
        'pp2w': p['pp2']['w'], 'pp2b': row(p['pp2']['b']),
    }
    packed = jnp.concatenate(
        [jnp.pad(vals[nm], ((0, (-r) % 8), (0, 128 - c)))
         for nm, r, c in _SEGS], axis=0)
    cin = jnp.pad(
        jnp.concatenate([t.reshape(B, 1), topo_cond, stab_cond, sust_cond],
                        axis=1), ((0, 0), (0, 3)))

    node_pred, pos_pred = pl.pallas_call(
        _body,
        out_shape=[
            jax.ShapeDtypeStruct((N, 12), jnp.float32),
            jax.ShapeDtypeStruct((N, 3), jnp.float32),
        ],
    )(x, batch.reshape(N, 1), cin, packed)

    return node_pred, pos_pred


# speedup vs baseline: 1.4091x; 1.4091x over previous
import math

import jax
import jax.numpy as jnp
from jax.experimental import pallas as pl
from jax.experimental.pallas import tpu as pltpu

N = 10000
H = 64
TEMB = 64
B = 8

_LOG1E4 = math.log(10000.0)

_SEGS = (
    ('t1w', 64, 128), ('t1b', 1, 128), ('t2w', 128, 64), ('t2b', 1, 64),
    ('to1w', 7, 32), ('to1b', 1, 32), ('to2w', 32, 32), ('to2b', 1, 32),
    ('st1w', 2, 16), ('st1b', 1, 16), ('st2w', 16, 16), ('st2b', 1, 16),
    ('su1w', 3, 16), ('su1b', 1, 16), ('su2w', 16, 16), ('su2b', 1, 16),
    ('c1w', 64, 64), ('c1b', 1, 64), ('c2w', 64, 64), ('c2b', 1, 64),
    ('wv', 64, 128), ('wow', 128, 64), ('wob', 1, 64),
    ('new', 12, 64), ('neb', 1, 64),
    ('np1w', 64, 128), ('np1b', 1, 128), ('np2w', 128, 12), ('np2b', 1, 12),
    ('pp1w', 64, 64), ('pp1b', 1, 64), ('pp2w', 64, 3), ('pp2b', 1, 3),
)
_NW = len(_SEGS)
_NIN = 6 + _NW


def _silu(v):
    return v * jax.nn.sigmoid(v)


def _mm(a, b):
    return jax.lax.dot_general(a, b, (((1,), (0,)), ((), ())),
                               preferred_element_type=jnp.float32)


def _body(*refs):
    (x_ref, batch_ref, t_ref, topo_ref, stab_ref, sust_ref) = refs[:6]
    hbm = refs[6:_NIN]
    node_out_ref, pos_out_ref = refs[_NIN:_NIN + 2]
    vmem = refs[_NIN + 2:_NIN + 2 + _NW]
    sem = refs[_NIN + 2 + _NW]

    for src, dst in zip(hbm, vmem):
        pltpu.make_async_copy(src, dst, sem).start()
    for src, dst in zip(hbm, vmem):
        pltpu.make_async_copy(src, dst, sem).wait()
    seg = {nm: vmem[i] for i, (nm, _, _) in enumerate(_SEGS)}

    half = TEMB // 2
    freq = jnp.exp(jax.lax.broadcasted_iota(jnp.int32, (1, half), 1)
                   .astype(jnp.float32) * (-_LOG1E4 / (half - 1)))
    ang = t_ref[...] * freq
    s, c = jnp.sin(ang), jnp.cos(ang)
    te_h = _silu(_mm(s, seg['t1w'][:half, :]) + _mm(c, seg['t1w'][half:, :])
                 + seg['t1b'][...])
    temb = _mm(te_h, seg['t2w'][...]) + seg['t2b'][...]

    te = _mm(_silu(_mm(topo_ref[...], seg['to1w'][...]) + seg['to1b'][...]),
             seg['to2w'][...]) + seg['to2b'][...]
    se = _mm(_silu(_mm(stab_ref[...], seg['st1w'][...]) + seg['st1b'][...]),
             seg['st2w'][...]) + seg['st2b'][...]
    ue = _mm(_silu(_mm(sust_ref[...], seg['su1w'][...]) + seg['su1b'][...]),
             seg['su2w'][...]) + seg['su2b'][...]
    c1w = seg['c1w']
    ce_h = _silu(_mm(te, c1w[0:32, :]) + _mm(se, c1w[32:48, :])
                 + _mm(ue, c1w[48:64, :]) + seg['c1b'][...])
    cond = _mm(ce_h, seg['c2w'][...]) + seg['c2b'][...]

    attn = _mm(_mm(cond, seg['wv'][...]), seg['wow'][...]) + seg['wob'][...]
    delta = attn + _silu(temb)

    onehot = (batch_ref[...] ==
              jax.lax.broadcasted_iota(jnp.int32, (N, B), 1)
              ).astype(jnp.float32)
    u = _mm(onehot, delta)
    h = _mm(x_ref[...], seg['new'][...]) + seg['neb'][...]
    h = h + u
    h = h + u
    h = h + u
    h = h + u
    a = _silu(_mm(h, seg['np1w'][...]) + seg['np1b'][...])
    node_out_ref[...] = _mm(a, seg['np2w'][...]) + seg['np2b'][...]
    g = _silu(_mm(h, seg['pp1w'][...]) + seg['pp1b'][...])
    pos_out_ref[...] = _mm(g, seg['pp2w'][...]) + seg['pp2b'][...]


def kernel(x, edge_index, edge_attr, pos, t, topo_cond, stab_cond, sust_cond,
           batch, params):
    del edge_index, edge_attr, pos
    p = params
    row = lambda b: b.reshape(1, -1)
    vals = {
        't1w': p['time1']['w'], 't1b': row(p['time1']['b']),
        't2w': p['time2']['w'], 't2b': row(p['time2']['b']),
        'to1w': p['topo1']['w'], 'to1b': row(p['topo1']['b']),
        'to2w': p['topo2']['w'], 'to2b': row(p['topo2']['b']),
        'st1w': p['stab1']['w'], 'st1b': row(p['stab1']['b']),
        'st2w': p['stab2']['w'], 'st2b': row(p['stab2']['b']),
        'su1w': p['sust1']['w'], 'su1b': row(p['sust1']['b']),
        'su2w': p['sust2']['w'], 'su2b': row(p['sust2']['b']),
        'c1w': p['comb1']['w'], 'c1b': row(p['comb1']['b']),
        'c2w': p['comb2']['w'], 'c2b': row(p['comb2']['b']),
        'wv': p['Wv'], 'wow': p['Wo']['w'], 'wob': row(p['Wo']['b']),
        'new': p['node_emb']['w'], 'neb': row(p['node_emb']['b']),
        'np1w': p['np1']['w'], 'np1b': row(p['np1']['b']),
        'np2w': p['np2']['w'], 'np2b': row(p['np2']['b']),
        'pp1w': p['pp1']['w'], 'pp1b': row(p['pp1']['b']),
        'pp2w': p['pp2']['w'], 'pp2b': row(p['pp2']['b']),
    }
    vmem_spec = pl.BlockSpec(memory_space=pltpu.MemorySpace.VMEM)
    any_spec = pl.BlockSpec(memory_space=pl.ANY)

    node_pred, pos_pred = pl.pallas_call(
        _body,
        in_specs=[vmem_spec] * 6 + [any_spec] * _NW,
        out_specs=[vmem_spec, vmem_spec],
        out_shape=[
            jax.ShapeDtypeStruct((N, 12), jnp.float32),
            jax.ShapeDtypeStruct((N, 3), jnp.float32),
        ],
        scratch_shapes=(
            [pltpu.VMEM((r, c), jnp.float32) for _, r, c in _SEGS]
            + [pltpu.SemaphoreType.DMA]),
    )(x, batch.reshape(N, 1), t.reshape(B, 1),
      topo_cond, stab_cond, sust_cond,
      *[vals[nm] for nm, _, _ in _SEGS])

    return node_pred, pos_pred
